# baseline (speedup 1.0000x reference)
import jax
import jax.numpy as jnp
from jax import lax
from jax.experimental import pallas as pl
from jax.experimental.pallas import tpu as pltpu

_B, _L, _E = 16384, 50, 64
_LE = _L * _E
_BB = 2048
_NB = _B // _BB
_NSEM = 4


def _body(embed_hbm, out_hbm, sems):
    i = pl.program_id(0)
    slot = lax.rem(i, _NSEM)
    pltpu.make_async_copy(
        embed_hbm.at[pl.ds(i * _BB, _BB), :],
        out_hbm.at[pl.ds(i * _BB, _BB), :],
        sems.at[slot],
    ).start()

    @pl.when(i >= _NSEM - 1)
    def _drain():
        j = i - (_NSEM - 1)
        pltpu.make_async_copy(
            embed_hbm.at[pl.ds(j * _BB, _BB), :],
            out_hbm.at[pl.ds(j * _BB, _BB), :],
            sems.at[lax.rem(j, _NSEM)],
        ).wait()

    @pl.when(i == _NB - 1)
    def _epilogue():
        for k in range(1, _NSEM):
            j = _NB - 1 - (_NSEM - 1) + k
            pltpu.make_async_copy(
                embed_hbm.at[pl.ds(j * _BB, _BB), :],
                out_hbm.at[pl.ds(j * _BB, _BB), :],
                sems.at[lax.rem(j, _NSEM)],
            ).wait()


def kernel(session_embed, session_index, session_bias, position_bias, item_bias):
    embed2d = session_embed.reshape(_B, _LE)
    out2d = pl.pallas_call(
        _body,
        grid=(_NB,),
        in_specs=[pl.BlockSpec(memory_space=pltpu.MemorySpace.HBM)],
        out_specs=pl.BlockSpec(memory_space=pltpu.MemorySpace.HBM),
        out_shape=jax.ShapeDtypeStruct((_B, _LE), jnp.float32),
        scratch_shapes=[pltpu.SemaphoreType.DMA((_NSEM,))],
        compiler_params=pltpu.CompilerParams(
            dimension_semantics=("arbitrary",),
        ),
    )(embed2d)
    return out2d.reshape(_B, _L, _E)


# speedup vs baseline: 12.8581x; 12.8581x over previous
import jax
import jax.numpy as jnp
from jax import lax
from jax.experimental import pallas as pl
from jax.experimental.pallas import tpu as pltpu

_B, _L, _E = 16384, 50, 64
_S = 20
_LE = _L * _E
_BB = 512
_NB = _B // _BB
_NBUF = 2
_SPLIT = 5
_LB = _LE // _SPLIT


def _in_copies(i, embed_hbm, ebuf, isem, slot):
    return [
        pltpu.make_async_copy(
            embed_hbm.at[pl.ds(i * _BB, _BB), pl.ds(j * _LB, _LB)],
            ebuf.at[slot, :, pl.ds(j * _LB, _LB)],
            isem.at[slot],
        )
        for j in range(_SPLIT)
    ]


def _out_copies(i, out_hbm, obuf, osem, slot):
    return [
        pltpu.make_async_copy(
            obuf.at[slot, :, pl.ds(j * _LB, _LB)],
            out_hbm.at[pl.ds(i * _BB, _BB), pl.ds(j * _LB, _LB)],
            osem.at[slot],
        )
        for j in range(_SPLIT)
    ]


def _fused_body(idx_ref, table_ref, pos_ref, item_ref, embed_hbm, out_hbm,
                ebuf, obuf, isem, osem):
    i = pl.program_id(0)
    slot = lax.rem(i, _NBUF)

    @pl.when(i == 0)
    def _prologue():
        for k in range(_NBUF):
            for c in _in_copies(k, embed_hbm, ebuf, isem, k):
                c.start()

    for c in _in_copies(i, embed_hbm, ebuf, isem, slot):
        c.wait()

    idx = idx_ref[...]
    table = table_ref[...]
    s_iota = lax.broadcasted_iota(jnp.int32, (1, _S), 1)
    sb = jnp.sum(jnp.where(idx == s_iota, table, 0.0), axis=1, keepdims=True)

    @pl.when(i >= _NBUF)
    def _drain():
        for c in _out_copies(i - _NBUF, out_hbm, obuf, osem, slot):
            c.wait()

    obuf[slot] = ebuf[slot] + sb + (pos_ref[...] + item_ref[...])
    for c in _out_copies(i, out_hbm, obuf, osem, slot):
        c.start()

    @pl.when(i + _NBUF < _NB)
    def _refill():
        for c in _in_copies(i + _NBUF, embed_hbm, ebuf, isem, slot):
            c.start()

    @pl.when(i == _NB - 1)
    def _epilogue():
        for k in range(1, _NBUF + 1):
            j = _NB - k
            for c in _out_copies(j, out_hbm, obuf, osem, lax.rem(j, _NBUF)):
                c.wait()


def kernel(session_embed, session_index, session_bias, position_bias, item_bias):
    embed2d = session_embed.reshape(_B, _LE)
    idx2d = session_index.astype(jnp.int32).reshape(_B, 1)
    table = session_bias.reshape(1, _S)
    pos2d = jnp.broadcast_to(position_bias, (1, _L, _E)).reshape(1, _LE)
    item2d = jnp.broadcast_to(item_bias, (1, _L, _E)).reshape(1, _LE)

    out2d = pl.pallas_call(
        _fused_body,
        grid=(_NB,),
        in_specs=[
            pl.BlockSpec((_BB, 1), lambda i: (i, 0)),
            pl.BlockSpec((1, _S), lambda i: (0, 0)),
            pl.BlockSpec((1, _LE), lambda i: (0, 0)),
            pl.BlockSpec((1, _LE), lambda i: (0, 0)),
            pl.BlockSpec(memory_space=pltpu.MemorySpace.HBM),
        ],
        out_specs=pl.BlockSpec(memory_space=pltpu.MemorySpace.HBM),
        out_shape=jax.ShapeDtypeStruct((_B, _LE), jnp.float32),
        scratch_shapes=[
            pltpu.VMEM((_NBUF, _BB, _LE), jnp.float32),
            pltpu.VMEM((_NBUF, _BB, _LE), jnp.float32),
            pltpu.SemaphoreType.DMA((_NBUF,)),
            pltpu.SemaphoreType.DMA((_NBUF,)),
        ],
        compiler_params=pltpu.CompilerParams(
            dimension_semantics=("arbitrary",),
        ),
    )(idx2d, table, pos2d, item2d, embed2d)
    return out2d.reshape(_B, _L, _E)
